# Initial kernel scaffold; baseline (speedup 1.0000x reference)
import jax
import jax.numpy as jnp
from jax.experimental import pallas as pl


def kernel(token_embeddings, type_indices, type_table, ln_weight, ln_bias):
    raise NotImplementedError("write your pallas kernel here")


# speedup vs baseline: 3.2241x; 3.2241x over previous
import jax
import jax.numpy as jnp
from jax.experimental import pallas as pl
from jax.experimental.pallas import tpu as pltpu

_EMBED = 1024
_TPAD = 16
_EPS = 1e-5
_BLOCK = 1024


def _fused_body(idx_ref, tok_ref, tab_ref, w_ref, b_ref, out_ref):
    tok = tok_ref[...]
    ids = idx_ref[...]
    iota = jax.lax.broadcasted_iota(jnp.int32, (tok.shape[0], _TPAD), 1)
    onehot = (ids == iota).astype(jnp.float32)
    emb = jnp.dot(onehot, tab_ref[...],
                  preferred_element_type=jnp.float32)
    x = tok + emb
    mean = jnp.mean(x, axis=-1, keepdims=True)
    ex2 = jnp.mean(x * x, axis=-1, keepdims=True)
    var = ex2 - mean * mean
    inv = jax.lax.rsqrt(var + _EPS)
    y = (x - mean) * inv
    out_ref[...] = y * w_ref[...] + b_ref[...]


def kernel(token_embeddings, type_indices, type_table, ln_weight, ln_bias):
    seq, embed = token_embeddings.shape
    ntypes = type_table.shape[0]
    ids = type_indices.astype(jnp.int32).reshape(seq, 1)
    tab = jnp.zeros((_TPAD, embed), jnp.float32).at[:ntypes].set(type_table)
    w = ln_weight.reshape(1, embed)
    b = ln_bias.reshape(1, embed)

    out = pl.pallas_call(
        _fused_body,
        grid=(seq // _BLOCK,),
        in_specs=[
            pl.BlockSpec((_BLOCK, 1), lambda i: (i, 0)),
            pl.BlockSpec((_BLOCK, embed), lambda i: (i, 0)),
            pl.BlockSpec((_TPAD, embed), lambda i: (0, 0)),
            pl.BlockSpec((1, embed), lambda i: (0, 0)),
            pl.BlockSpec((1, embed), lambda i: (0, 0)),
        ],
        out_specs=pl.BlockSpec((_BLOCK, embed), lambda i: (i, 0)),
        out_shape=jax.ShapeDtypeStruct((seq, embed), jnp.float32),
    )(ids, token_embeddings, tab, w, b)
    return out[None, :, :]
